# Initial kernel scaffold; baseline (speedup 1.0000x reference)
import jax
import jax.numpy as jnp
from jax.experimental import pallas as pl


def kernel(query_pts, voxel_features_0, voxel_features_1, hash2voxel_0, hash2voxel_1):
    raise NotImplementedError("write your pallas kernel here")


# speedup vs baseline: 20.6445x; 20.6445x over previous
import functools

import jax
import jax.numpy as jnp
from jax import lax
from jax._src import config as _jax_config
from jax.experimental import pallas as pl
from jax.experimental.pallas import tpu as pltpu
from jax.experimental.pallas import tpu_sc as plsc

_RES0 = 0.06
_SCALE = 2.0
_HMASK = 2097152 - 1
_P0, _P1, _P2 = 73856093, 19349669, 83492791
_FDIM = 32
_NW = 32
_CH = 128
_L = 16


def _corner_hash_and_weights(q_v, hv_v, w_v, res):
    inv = jnp.float32(res)
    for g in range(_CH // _L):
        sl = pl.ds(g * _L, _L)
        sx = q_v[0, sl] / inv
        sy = q_v[1, sl] / inv
        sz = q_v[2, sl] / inv
        bx = sx.astype(jnp.int32)
        by = sy.astype(jnp.int32)
        bz = sz.astype(jnp.int32)
        fx = sx - bx.astype(jnp.float32)
        fy = sy - by.astype(jnp.float32)
        fz = sz - bz.astype(jnp.float32)
        one = jnp.float32(1.0)
        for c in range(8):
            ox, oy, oz = (c >> 2) & 1, (c >> 1) & 1, c & 1
            h = ((bx + ox) * _P0 + (by + oy) * _P1 + (bz + oz) * _P2)
            hv_v[c, sl] = h & _HMASK
            wx = fx if ox else one - fx
            wy = fy if oy else one - fy
            wz = fz if oz else one - fz
            w_v[pl.ds(c * _CH + g * _L, _L)] = wx * wy * wz


def _level(feats, h2v, res, lvl, q_v, hv_v, vi_v, w_v, rows_v, out_v, sem):
    _corner_hash_and_weights(q_v, hv_v, w_v, res)
    cps = [pltpu.async_copy(h2v.at[hv_v.at[c]], vi_v.at[c], sem)
           for c in range(8)]
    for cp in cps:
        cp.wait()
    for c in range(8):
        for g in range(_CH // _L):
            sl = pl.ds(g * _L, _L)
            v = vi_v[c, sl]
            ws = pl.ds(c * _CH + g * _L, _L)
            w_v[ws] = jnp.where(v >= 0, w_v[ws], jnp.float32(0.0))
            vi_v[c, sl] = jnp.maximum(v, 0)
    cps = [pltpu.async_copy(feats.at[vi_v.at[c]], rows_v.at[c], sem)
           for c in range(8)]
    for cp in cps:
        cp.wait()

    def blend(q, carry):
        qi = jnp.full((_L,), q, jnp.int32)

        def wbcast(c):
            return plsc.load_gather(
                w_v, [jnp.full((_L,), c * _CH, jnp.int32) + qi])

        w0 = wbcast(0)
        a0 = w0 * rows_v[0, q, pl.ds(0, _L)]
        a1 = w0 * rows_v[0, q, pl.ds(_L, _L)]
        for c in range(1, 8):
            wc = wbcast(c)
            a0 = a0 + wc * rows_v[c, q, pl.ds(0, _L)]
            a1 = a1 + wc * rows_v[c, q, pl.ds(_L, _L)]
        out_v[q, pl.ds(lvl * _FDIM, _L)] = a0
        out_v[q, pl.ds(lvl * _FDIM + _L, _L)] = a1
        return carry

    lax.fori_loop(jnp.int32(0), jnp.int32(_CH), blend, 0)


def _body(qT, f0, h0, f1, h1, out, q_v, hv_v, vi_v, w_v, rows_v, out_v, sem):
    wid = (lax.axis_index("s").astype(jnp.int32) * jnp.int32(2)
           + lax.axis_index("c").astype(jnp.int32))
    nq_w = qT.shape[1] // _NW
    nchunks = nq_w // _CH

    def chunk(i, carry):
        qbase = wid * jnp.int32(nq_w) + i * jnp.int32(_CH)
        pltpu.sync_copy(qT.at[:, pl.ds(qbase, _CH)], q_v)
        _level(f0, h0, _RES0, 0, q_v, hv_v, vi_v, w_v, rows_v, out_v, sem)
        _level(f1, h1, _RES0 * _SCALE, 1,
               q_v, hv_v, vi_v, w_v, rows_v, out_v, sem)
        pltpu.sync_copy(out_v, out.at[pl.ds(qbase, _CH)])
        return carry

    lax.fori_loop(jnp.int32(0), jnp.int32(nchunks), chunk, 0)


def _run(qT, f0, h0, f1, h1):
    m = qT.shape[1]
    grid_kernel = functools.partial(
        pl.kernel,
        out_type=jax.ShapeDtypeStruct((m, 2 * _FDIM), jnp.float32),
        mesh=plsc.VectorSubcoreMesh(core_axis_name="c", subcore_axis_name="s"),
        scratch_types=[
            pltpu.VMEM((3, _CH), jnp.float32),
            pltpu.VMEM((8, _CH), jnp.int32),
            pltpu.VMEM((8, _CH), jnp.int32),
            pltpu.VMEM((8 * _CH,), jnp.float32),
            pltpu.VMEM((8, _CH, _FDIM), jnp.float32),
            pltpu.VMEM((_CH, 2 * _FDIM), jnp.float32),
            pltpu.SemaphoreType.DMA,
        ],
        compiler_params=pltpu.CompilerParams(
            needs_layout_passes=False, use_tc_tiling_on_sc=False),
    )(_body)
    return grid_kernel(qT, f0, h0, f1, h1)


def kernel(query_pts, voxel_features_0, voxel_features_1,
           hash2voxel_0, hash2voxel_1):
    qT = query_pts.astype(jnp.float32).T
    f0 = voxel_features_0.astype(jnp.float32)
    f1 = voxel_features_1.astype(jnp.float32)
    h0 = hash2voxel_0.astype(jnp.int32)
    h1 = hash2voxel_1.astype(jnp.int32)
    with _jax_config.enable_x64(False):
        return _run(qT, f0, h0, f1, h1)


# speedup vs baseline: 25.5814x; 1.2391x over previous
import functools

import jax
import jax.numpy as jnp
from jax import lax
from jax._src import config as _jax_config
from jax.experimental import pallas as pl
from jax.experimental.pallas import tpu as pltpu
from jax.experimental.pallas import tpu_sc as plsc

_RES0 = 0.06
_SCALE = 2.0
_HMASK = 2097152 - 1
_P0, _P1, _P2 = 73856093, 19349669, 83492791
_FDIM = 32
_NW = 32
_CH = 128
_L = 16


def _hash_w(q_v, hv, wraw, res):
    inv = jnp.float32(res)
    for g in range(_CH // _L):
        sl = pl.ds(g * _L, _L)
        sx = q_v[0, sl] / inv
        sy = q_v[1, sl] / inv
        sz = q_v[2, sl] / inv
        bx = sx.astype(jnp.int32)
        by = sy.astype(jnp.int32)
        bz = sz.astype(jnp.int32)
        fx = sx - bx.astype(jnp.float32)
        fy = sy - by.astype(jnp.float32)
        fz = sz - bz.astype(jnp.float32)
        one = jnp.float32(1.0)
        for c in range(8):
            ox, oy, oz = (c >> 2) & 1, (c >> 1) & 1, c & 1
            h = ((bx + ox) * _P0 + (by + oy) * _P1 + (bz + oz) * _P2)
            hv[c, sl] = h & _HMASK
            wx = fx if ox else one - fx
            wy = fy if oy else one - fy
            wz = fz if oz else one - fz
            wraw[pl.ds(c * _CH + g * _L, _L)] = wx * wy * wz


def _validity(viraw, vvi, wraw, wq):
    for c in range(8):
        for g in range(_CH // _L):
            sl = pl.ds(g * _L, _L)
            ws = pl.ds(c * _CH + g * _L, _L)
            v = viraw[c, sl]
            wq[ws] = jnp.where(v >= 0, wraw[ws], jnp.float32(0.0))
            vvi[c, sl] = jnp.maximum(v, 0)


def _fire_v(h2v, hv, viraw, sem):
    return [pltpu.async_copy(h2v.at[hv.at[c]], viraw.at[c], sem)
            for c in range(8)]


def _wait_v(h2v, hv, viraw, sem):
    for c in range(8):
        pltpu.make_async_copy(h2v.at[hv.at[c]], viraw.at[c], sem).wait()


def _fire_rows(feats, vvi, rows, sem):
    return [pltpu.async_copy(feats.at[vvi.at[c]], rows.at[c], sem)
            for c in range(8)]


def _wait_rows(feats, vvi, rows, sem):
    for c in range(8):
        pltpu.make_async_copy(feats.at[vvi.at[c]], rows.at[c], sem).wait()


def _blend(rows, wq, out_v, lvl):
    def body(q, carry):
        qi = jnp.full((_L,), q, jnp.int32)

        def wbcast(c):
            return plsc.load_gather(
                wq, [jnp.full((_L,), c * _CH, jnp.int32) + qi])

        w0 = wbcast(0)
        a0 = w0 * rows[0, q, pl.ds(0, _L)]
        a1 = w0 * rows[0, q, pl.ds(_L, _L)]
        for c in range(1, 8):
            wc = wbcast(c)
            a0 = a0 + wc * rows[c, q, pl.ds(0, _L)]
            a1 = a1 + wc * rows[c, q, pl.ds(_L, _L)]
        out_v[q, pl.ds(lvl * _FDIM, _L)] = a0
        out_v[q, pl.ds(lvl * _FDIM + _L, _L)] = a1
        return carry

    lax.fori_loop(0, _CH, body, 0, unroll=2)


def _body(qT, f0, h0, f1, h1, out,
          q_v, hv0, hv1, viraw0, viraw1, vvi0, vvi1,
          wraw0, wraw1, wq0, wq1, rows0, rows1, out_v,
          sem_va, sem_vb, sem_ra, sem_rb):
    wid = (lax.axis_index("s").astype(jnp.int32) * jnp.int32(2)
           + lax.axis_index("c").astype(jnp.int32))
    nq_w = qT.shape[1] // _NW
    nchunks = nq_w // _CH
    res1 = _RES0 * _SCALE

    base0 = wid * jnp.int32(nq_w)
    pltpu.sync_copy(qT.at[:, pl.ds(base0, _CH)], q_v)
    _hash_w(q_v, hv0, wraw0, _RES0)
    _fire_v(h0, hv0, viraw0, sem_va)

    def chunk(t, carry):
        qbase = base0 + t * jnp.int32(_CH)
        _wait_v(h0, hv0, viraw0, sem_va)
        _validity(viraw0, vvi0, wraw0, wq0)
        _fire_rows(f0, vvi0, rows0, sem_ra)
        _hash_w(q_v, hv1, wraw1, res1)
        _fire_v(h1, hv1, viraw1, sem_vb)

        @pl.when(t > jnp.int32(0))
        def _():
            _wait_rows(f1, vvi1, rows1, sem_rb)
            _blend(rows1, wq1, out_v, 1)
            pltpu.sync_copy(out_v, out.at[pl.ds(qbase - jnp.int32(_CH), _CH)])

        _wait_v(h1, hv1, viraw1, sem_vb)
        _validity(viraw1, vvi1, wraw1, wq1)
        _fire_rows(f1, vvi1, rows1, sem_rb)

        @pl.when(t < jnp.int32(nchunks - 1))
        def _():
            pltpu.sync_copy(qT.at[:, pl.ds(qbase + jnp.int32(_CH), _CH)], q_v)
            _hash_w(q_v, hv0, wraw0, _RES0)
            _fire_v(h0, hv0, viraw0, sem_va)

        _wait_rows(f0, vvi0, rows0, sem_ra)
        _blend(rows0, wq0, out_v, 0)
        return carry

    lax.fori_loop(jnp.int32(0), jnp.int32(nchunks), chunk, 0)

    _wait_rows(f1, vvi1, rows1, sem_rb)
    _blend(rows1, wq1, out_v, 1)
    lastbase = base0 + jnp.int32((nchunks - 1) * _CH)
    pltpu.sync_copy(out_v, out.at[pl.ds(lastbase, _CH)])


def _run(qT, f0, h0, f1, h1):
    m = qT.shape[1]
    grid_kernel = functools.partial(
        pl.kernel,
        out_type=jax.ShapeDtypeStruct((m, 2 * _FDIM), jnp.float32),
        mesh=plsc.VectorSubcoreMesh(core_axis_name="c", subcore_axis_name="s"),
        scratch_types=[
            pltpu.VMEM((3, _CH), jnp.float32),
            pltpu.VMEM((8, _CH), jnp.int32),
            pltpu.VMEM((8, _CH), jnp.int32),
            pltpu.VMEM((8, _CH), jnp.int32),
            pltpu.VMEM((8, _CH), jnp.int32),
            pltpu.VMEM((8, _CH), jnp.int32),
            pltpu.VMEM((8, _CH), jnp.int32),
            pltpu.VMEM((8 * _CH,), jnp.float32),
            pltpu.VMEM((8 * _CH,), jnp.float32),
            pltpu.VMEM((8 * _CH,), jnp.float32),
            pltpu.VMEM((8 * _CH,), jnp.float32),
            pltpu.VMEM((8, _CH, _FDIM), jnp.float32),
            pltpu.VMEM((8, _CH, _FDIM), jnp.float32),
            pltpu.VMEM((_CH, 2 * _FDIM), jnp.float32),
            pltpu.SemaphoreType.DMA,
            pltpu.SemaphoreType.DMA,
            pltpu.SemaphoreType.DMA,
            pltpu.SemaphoreType.DMA,
        ],
        compiler_params=pltpu.CompilerParams(
            needs_layout_passes=False, use_tc_tiling_on_sc=False),
    )(_body)
    return grid_kernel(qT, f0, h0, f1, h1)


def kernel(query_pts, voxel_features_0, voxel_features_1,
           hash2voxel_0, hash2voxel_1):
    qT = query_pts.astype(jnp.float32).T
    f0 = voxel_features_0.astype(jnp.float32)
    f1 = voxel_features_1.astype(jnp.float32)
    h0 = hash2voxel_0.astype(jnp.int32)
    h1 = hash2voxel_1.astype(jnp.int32)
    with _jax_config.enable_x64(False):
        return _run(qT, f0, h0, f1, h1)
